# baseline (speedup 1.0000x reference)
import functools

import jax
import jax.numpy as jnp
from jax import lax
from jax.experimental import pallas as pl
from jax.experimental.pallas import tpu as pltpu
from jax.experimental.pallas import tpu_sc as plsc

N_NODES = 10000
D_IN = 128
D_OUT = 128
N_EDGES = 320000

DE = 144
NC = 2
NS = 16
NW = NC * NS
CHUNK = 128
CH_PER_W = 80
E_PER_W = CH_PER_W * CHUNK
E_PAD = NW * E_PER_W
ACC_ROWS = 10240
DUMP_ROW = N_NODES
ROWS_PER_TILE = ACC_ROWS // NS

_sc_mesh = plsc.VectorSubcoreMesh(
    core_axis_name="c", subcore_axis_name="s", num_cores=NC, num_subcores=NS)


@functools.partial(
    pl.kernel,
    out_type=jax.ShapeDtypeStruct((NC, ACC_ROWS, DE), jnp.float32),
    mesh=_sc_mesh,
    compiler_params=pltpu.CompilerParams(use_tc_tiling_on_sc=False),
    scratch_types=[
        pltpu.VMEM((2, CHUNK), jnp.int32),
        pltpu.VMEM((2, CHUNK), jnp.int32),
        pltpu.VMEM((CHUNK, DE), jnp.float32),
        pltpu.VMEM((CHUNK, DE), jnp.float32),
        pltpu.SemaphoreType.DMA,
        pltpu.SemaphoreType.DMA,
        pltpu.VMEM_SHARED((ACC_ROWS, DE), jnp.float32),
    ],
)
def _sc_aggregate(featext_hbm, edges_hbm, zeros_hbm, out_hbm,
                  ib0, ib1, rows0, rows1, gsem, isem, acc_sh):
    c = lax.axis_index("c")
    s = lax.axis_index("s")
    wid = s * NC + c
    half = CH_PER_W // 2

    def wait_rows(buf):
        pltpu.make_async_copy(featext_hbm.at[ib0.at[0]], buf, gsem).wait()

    def wait_idx(buf):
        pltpu.make_async_copy(edges_hbm.at[wid, 0], buf, isem).wait()

    pltpu.sync_copy(zeros_hbm, acc_sh.at[pl.ds(s * ROWS_PER_TILE, ROWS_PER_TILE)])
    plsc.subcore_barrier()

    pltpu.sync_copy(edges_hbm.at[wid, 0], ib0)
    pltpu.async_copy(featext_hbm.at[ib0.at[0]], rows0, gsem)
    pltpu.async_copy(edges_hbm.at[wid, 1], ib1, isem)

    def body(j2, carry):
        j = 2 * j2
        wait_rows(rows0)
        wait_idx(ib1)
        pltpu.async_copy(featext_hbm.at[ib1.at[0]], rows1, gsem)
        pltpu.sync_copy(rows0, acc_sh.at[ib0.at[1]], add=True)

        @pl.when(j + 2 < CH_PER_W)
        def _():
            pltpu.async_copy(edges_hbm.at[wid, j + 2], ib0, isem)

        wait_rows(rows1)

        @pl.when(j + 2 < CH_PER_W)
        def _():
            wait_idx(ib0)
            pltpu.async_copy(featext_hbm.at[ib0.at[0]], rows0, gsem)

        pltpu.sync_copy(rows1, acc_sh.at[ib1.at[1]], add=True)

        @pl.when(j + 3 < CH_PER_W)
        def _():
            pltpu.async_copy(edges_hbm.at[wid, j + 3], ib1, isem)

        return carry

    lax.fori_loop(0, half, body, 0)
    plsc.subcore_barrier()

    pltpu.sync_copy(acc_sh.at[pl.ds(s * ROWS_PER_TILE, ROWS_PER_TILE)],
                    out_hbm.at[c, pl.ds(s * ROWS_PER_TILE, ROWS_PER_TILE)])


def _tc_combine_body(x_ref, p0_ref, p1_ref, ws_ref, wn_ref, b_ref, o_ref):
    x = x_ref[...]
    p = p0_ref[...] + p1_ref[...]
    neigh_sum = p[:, :D_IN]
    deg = p[:, D_IN:D_IN + 1]
    h_neigh = neigh_sum / jnp.maximum(deg, 1.0)
    dn = (((1,), (1,)), ((), ()))
    out = lax.dot_general(x, ws_ref[...], dn, preferred_element_type=jnp.float32)
    out += lax.dot_general(h_neigh, wn_ref[...], dn, preferred_element_type=jnp.float32)
    o_ref[...] = out + b_ref[...]


def _tc_combine(feat, p0, p1, w_self, w_neigh, b2):
    blk = 1000
    grid = N_NODES // blk
    return pl.pallas_call(
        _tc_combine_body,
        grid=(grid,),
        in_specs=[
            pl.BlockSpec((blk, D_IN), lambda i: (i, 0)),
            pl.BlockSpec((blk, DE), lambda i: (i, 0)),
            pl.BlockSpec((blk, DE), lambda i: (i, 0)),
            pl.BlockSpec((D_OUT, D_IN), lambda i: (0, 0)),
            pl.BlockSpec((D_OUT, D_IN), lambda i: (0, 0)),
            pl.BlockSpec((1, D_OUT), lambda i: (0, 0)),
        ],
        out_specs=pl.BlockSpec((blk, D_OUT), lambda i: (i, 0)),
        out_shape=jax.ShapeDtypeStruct((N_NODES, D_OUT), jnp.float32),
    )(feat, p0, p1, w_self, w_neigh, b2)


def kernel(feat, edge_index, W_self, W_neigh, b_neigh):
    ones = jnp.ones((N_NODES, 1), jnp.float32)
    pad_cols = jnp.zeros((N_NODES, DE - D_IN - 1), jnp.float32)
    feat_ext = jnp.concatenate([feat, ones, pad_cols], axis=1)

    n_pad = E_PAD - N_EDGES
    src_p = jnp.concatenate([edge_index[0], jnp.zeros((n_pad,), jnp.int32)])
    dst_p = jnp.concatenate([edge_index[1],
                             jnp.full((n_pad,), DUMP_ROW, jnp.int32)])
    edges_p = jnp.stack([src_p.reshape(NW, CH_PER_W, CHUNK),
                         dst_p.reshape(NW, CH_PER_W, CHUNK)], axis=2)
    zeros_tile = jnp.zeros((ROWS_PER_TILE, DE), jnp.float32)

    partials = _sc_aggregate(feat_ext, edges_p, zeros_tile)
    b2 = b_neigh.reshape(1, D_OUT)
    return _tc_combine(feat, partials[0], partials[1], W_self, W_neigh, b2)
